# baseline (speedup 1.0000x reference)
import jax
import jax.numpy as jnp
from jax import lax
from jax.experimental import pallas as pl
from jax.experimental.pallas import tpu as pltpu
from jax.experimental.pallas import tpu_sc as plsc

DIM = 1024
T = 20
LANES = 16
CHUNKS = DIM // LANES
NEWTON_ITERS = 3
EPS = 1e-5
NSLOT = 8
PAIRS = DIM // 32

_GATHER_DN = lax.GatherDimensionNumbers(
    offset_dims=(), collapsed_slice_dims=(0,), start_index_map=(0,))


def _lane_shuffle(v, idx):
    return lax.gather(v, idx[:, None], _GATHER_DN, slice_sizes=(1,),
                      mode=lax.GatherScatterMode.PROMISE_IN_BOUNDS)


def _lane_allreduce_sum(v):
    lanes = lax.iota(jnp.int32, LANES)
    for shift in (1, 2, 4, 8):
        v = v + _lane_shuffle(v, lanes ^ shift)
    return v


def _rsqrt16(a):
    xi = lax.bitcast_convert_type(a, jnp.int32)
    yi = jnp.int32(0x5F3759DF) - (xi >> 1)
    y = lax.bitcast_convert_type(yi, jnp.float32)
    half = a * 0.5
    for _ in range(NEWTON_ITERS):
        y = y * (1.5 - half * y * y)
    return y


def _make_sc_kernel(num_bags, bags_per_w):
    mesh = plsc.VectorSubcoreMesh(core_axis_name="c", subcore_axis_name="s")
    nc = mesh.num_cores

    def run(idx, table):
        @pl.kernel(
            out_type=jax.ShapeDtypeStruct((num_bags, DIM), jnp.float32),
            mesh=mesh,
            scratch_types=[
                pltpu.VMEM((bags_per_w, T), jnp.int32),
                pltpu.VMEM((NSLOT, T, DIM // 2), jnp.int32),
                pltpu.VMEM((NSLOT, DIM), jnp.float32),
            ] + [pltpu.SemaphoreType.DMA] * (2 * NSLOT),
            compiler_params=pltpu.CompilerParams(use_tc_tiling_on_sc=False),
        )
        def body(idx_hbm, table_hbm, out_hbm, idx_v, rows_v, row_v, *sems):
            gsems = sems[:NSLOT]
            osems = sems[NSLOT:]
            wid = lax.axis_index("s") * nc + lax.axis_index("c")
            base = wid * bags_per_w
            pltpu.sync_copy(idx_hbm.at[pl.ds(base, bags_per_w)], idx_v)
            zeros = jnp.zeros((LANES,), jnp.float32)

            for s in range(NSLOT - 1):
                pltpu.async_copy(
                    table_hbm.at[idx_v.at[s]], rows_v.at[s], gsems[s])

            def do_bag(j, slot):
                pltpu.make_async_copy(
                    table_hbm.at[idx_v.at[j]], rows_v.at[slot],
                    gsems[slot]).wait()

                @pl.when(j >= NSLOT)
                def _():
                    pltpu.make_async_copy(
                        row_v.at[slot], out_hbm.at[base + j - NSLOT],
                        osems[slot]).wait()

                nslot = (slot + NSLOT - 1) % NSLOT
                @pl.when(j + NSLOT - 1 < bags_per_w)
                def _():
                    pltpu.async_copy(
                        table_hbm.at[idx_v.at[j + NSLOT - 1]],
                        rows_v.at[nslot], gsems[nslot])

                def tree(vals):
                    while len(vals) > 1:
                        nxt = [vals[k] + vals[k + 1]
                               for k in range(0, len(vals) - 1, 2)]
                        if len(vals) % 2:
                            nxt[-1] = nxt[-1] + vals[-1]
                        vals = nxt
                    return vals[0]

                def widen(t, c):
                    w = rows_v[slot, t, pl.ds(c * LANES, LANES)]
                    even = lax.bitcast_convert_type(w << 16, jnp.float32)
                    odd = lax.bitcast_convert_type(
                        w & jnp.int32(-65536), jnp.float32)
                    return even, odd

                def chunk_body(c, carry):
                    vsum, vsq = carry
                    vals = [widen(t, c) for t in range(T)]
                    a = tree([v[0] for v in vals])
                    b = tree([v[1] for v in vals])
                    row_v[slot, pl.ds(c * 32, LANES)] = a
                    row_v[slot, pl.ds(c * 32 + LANES, LANES)] = b
                    return (vsum + (a + b), vsq + (a * a + b * b))

                vsum, vsq = lax.fori_loop(
                    0, PAIRS, chunk_body, (zeros, zeros), unroll=2)
                mean = _lane_allreduce_sum(vsum) * (1.0 / DIM)
                ex2 = _lane_allreduce_sum(vsq) * (1.0 / DIM)
                rstd = _rsqrt16(ex2 - mean * mean + EPS)

                lanes = lax.iota(jnp.int32, LANES)
                ilo = lanes >> 1
                ihi = ilo + (LANES // 2)
                even_mask = (lanes & 1) == 0

                def norm_body(c, _):
                    a = (row_v[slot, pl.ds(c * 32, LANES)] - mean) * rstd
                    b = (row_v[slot, pl.ds(c * 32 + LANES, LANES)]
                         - mean) * rstd
                    c0 = jnp.where(even_mask, _lane_shuffle(a, ilo),
                                   _lane_shuffle(b, ilo))
                    c1 = jnp.where(even_mask, _lane_shuffle(a, ihi),
                                   _lane_shuffle(b, ihi))
                    row_v[slot, pl.ds(c * 32, LANES)] = c0
                    row_v[slot, pl.ds(c * 32 + LANES, LANES)] = c1
                    return 0

                lax.fori_loop(0, PAIRS, norm_body, 0)
                pltpu.async_copy(row_v.at[slot], out_hbm.at[base + j],
                                 osems[slot])

            @pl.loop(0, bags_per_w, step=NSLOT)
            def _(i):
                for b in range(NSLOT):
                    do_bag(i + b, b)

            for s in range(NSLOT):
                j = bags_per_w - NSLOT + s
                pltpu.make_async_copy(
                    row_v.at[s], out_hbm.at[base + j], osems[s]).wait()

        return body(idx, table)

    return run


_NUM_WORKERS = 32
_sc_run = None


def kernel(x, table):
    global _sc_run
    B, F, t = x.shape
    num_bags = B * F
    if _sc_run is None:
        _sc_run = _make_sc_kernel(num_bags, num_bags // _NUM_WORKERS)
    idx = x.reshape(num_bags, t)
    e16 = jax.lax.bitcast_convert_type(
        table[:, 0::2].astype(jnp.bfloat16), jnp.uint16)
    o16 = jax.lax.bitcast_convert_type(
        table[:, 1::2].astype(jnp.bfloat16), jnp.uint16)
    tb = (o16.astype(jnp.int32) << 16) | e16.astype(jnp.int32)
    out = _sc_run(idx, tb)
    return out.reshape(B, F, table.shape[1])


# speedup vs baseline: 5.7220x; 5.7220x over previous
import jax
import jax.numpy as jnp
from jax import lax
from jax.experimental import pallas as pl
from jax.experimental.pallas import tpu as pltpu
from jax.experimental.pallas import tpu_sc as plsc

DIM = 1024
T = 20
LANES = 16
CHUNKS = DIM // LANES
NEWTON_ITERS = 3
EPS = 1e-5
NSLOT = 4

_GATHER_DN = lax.GatherDimensionNumbers(
    offset_dims=(), collapsed_slice_dims=(0,), start_index_map=(0,))


def _lane_shuffle(v, idx):
    return lax.gather(v, idx[:, None], _GATHER_DN, slice_sizes=(1,),
                      mode=lax.GatherScatterMode.PROMISE_IN_BOUNDS)


def _lane_allreduce_sum(v):
    lanes = lax.iota(jnp.int32, LANES)
    for shift in (1, 2, 4, 8):
        v = v + _lane_shuffle(v, lanes ^ shift)
    return v


def _rsqrt16(a):
    xi = lax.bitcast_convert_type(a, jnp.int32)
    yi = jnp.int32(0x5F3759DF) - (xi >> 1)
    y = lax.bitcast_convert_type(yi, jnp.float32)
    half = a * 0.5
    for _ in range(NEWTON_ITERS):
        y = y * (1.5 - half * y * y)
    return y


def _make_sc_kernel(num_bags, bags_per_w):
    mesh = plsc.VectorSubcoreMesh(core_axis_name="c", subcore_axis_name="s")
    nc = mesh.num_cores

    def run(idx, table):
        @pl.kernel(
            out_type=jax.ShapeDtypeStruct((num_bags, DIM), jnp.float32),
            mesh=mesh,
            scratch_types=[
                pltpu.VMEM((bags_per_w, T), jnp.int32),
                pltpu.VMEM((NSLOT, T, DIM), jnp.float32),
                pltpu.VMEM((NSLOT, DIM), jnp.float32),
            ] + [pltpu.SemaphoreType.DMA] * (2 * NSLOT),
            compiler_params=pltpu.CompilerParams(use_tc_tiling_on_sc=False),
        )
        def body(idx_hbm, table_hbm, out_hbm, idx_v, rows_v, row_v, *sems):
            gsems = sems[:NSLOT]
            osems = sems[NSLOT:]
            wid = lax.axis_index("s") * nc + lax.axis_index("c")
            base = wid * bags_per_w
            pltpu.sync_copy(idx_hbm.at[pl.ds(base, bags_per_w)], idx_v)
            zeros = jnp.zeros((LANES,), jnp.float32)

            for s in range(NSLOT - 1):
                pltpu.async_copy(
                    table_hbm.at[idx_v.at[s]], rows_v.at[s], gsems[s])

            def do_bag(j, slot):
                pltpu.make_async_copy(
                    table_hbm.at[idx_v.at[j]], rows_v.at[slot],
                    gsems[slot]).wait()

                @pl.when(j >= NSLOT)
                def _():
                    pltpu.make_async_copy(
                        row_v.at[slot], out_hbm.at[base + j - NSLOT],
                        osems[slot]).wait()

                nslot = (slot + NSLOT - 1) % NSLOT
                @pl.when(j + NSLOT - 1 < bags_per_w)
                def _():
                    pltpu.async_copy(
                        table_hbm.at[idx_v.at[j + NSLOT - 1]],
                        rows_v.at[nslot], gsems[nslot])

                def chunk_body(c, carry):
                    vsum, vsq = carry
                    vals = [rows_v[slot, t, pl.ds(c * LANES, LANES)]
                            for t in range(T)]
                    while len(vals) > 1:
                        nxt = [vals[k] + vals[k + 1]
                               for k in range(0, len(vals) - 1, 2)]
                        if len(vals) % 2:
                            nxt[-1] = nxt[-1] + vals[-1]
                        vals = nxt
                    s = vals[0]
                    row_v[slot, pl.ds(c * LANES, LANES)] = s
                    return (vsum + s, vsq + s * s)

                vsum, vsq = lax.fori_loop(
                    0, CHUNKS, chunk_body, (zeros, zeros), unroll=2)
                mean = _lane_allreduce_sum(vsum) * (1.0 / DIM)
                ex2 = _lane_allreduce_sum(vsq) * (1.0 / DIM)
                rstd = _rsqrt16(ex2 - mean * mean + EPS)

                def norm_body(c, _):
                    v = row_v[slot, pl.ds(c * LANES, LANES)]
                    row_v[slot, pl.ds(c * LANES, LANES)] = (v - mean) * rstd
                    return 0

                lax.fori_loop(0, CHUNKS, norm_body, 0)
                pltpu.async_copy(row_v.at[slot], out_hbm.at[base + j],
                                 osems[slot])

            @pl.loop(0, bags_per_w, step=NSLOT)
            def _(i):
                for b in range(NSLOT):
                    do_bag(i + b, b)

            for s in range(NSLOT):
                j = bags_per_w - NSLOT + s
                pltpu.make_async_copy(
                    row_v.at[s], out_hbm.at[base + j], osems[s]).wait()

        return body(idx, table)

    return run


_NUM_WORKERS = 32
_sc_run = None


def kernel(x, table):
    global _sc_run
    B, F, t = x.shape
    num_bags = B * F
    if _sc_run is None:
        _sc_run = _make_sc_kernel(num_bags, num_bags // _NUM_WORKERS)
    idx = x.reshape(num_bags, t)
    out = _sc_run(idx, table)
    return out.reshape(B, F, table.shape[1])
